# baseline (speedup 1.0000x reference)
import functools

import jax
import jax.numpy as jnp
from jax import lax
from jax.experimental import pallas as pl
from jax.experimental.pallas import tpu as pltpu
from jax.experimental.pallas import tpu_sc as plsc

_B, _C, _H, _W = 1024, 256, 14, 14
_HW = _H * _W
_NW = 32
_EPW = _B // _NW
_SLAB = 4


def _sc_gather_body(mp_hbm, idx_hbm, out_hbm,
                    idx_v, gidx_v, horiz_v, mrow_v, bstage, sem):
    cid = lax.axis_index("c")
    sid = lax.axis_index("s")
    wid = sid * 2 + cid
    b0 = wid * _EPW
    pltpu.sync_copy(idx_hbm.at[pl.ds(b0, _EPW)], idx_v)
    iota = lax.iota(jnp.int32, 16)
    for t in range(_EPW // 16):
        v = idx_v[pl.ds(16 * t, 16)]
        gidx_v[pl.ds(16 * t, 16)] = lax.shift_right_logical(v, 1)
        horiz_v[pl.ds(16 * t, 16)] = jnp.bitwise_and(v, 1) * (_C // 4)
    pltpu.async_copy(mp_hbm.at[gidx_v], mrow_v, sem).wait()

    def sel(e, _):
        se = jnp.broadcast_to(e, (16,))
        he = plsc.load_gather(horiz_v, [se])
        for q in range(_C // 64):
            w = plsc.load_gather(mrow_v, [se, he + q * 16 + iota])
            bstage[e, pl.ds(q * 16, 16)] = w
        return 0

    lax.fori_loop(0, _EPW, sel, 0)
    pltpu.sync_copy(bstage, out_hbm.at[pl.ds(b0, _EPW)])


def _sc_gather_masks(mpair, idx):
    mesh = plsc.VectorSubcoreMesh(core_axis_name="c", subcore_axis_name="s")
    run = functools.partial(
        pl.kernel, mesh=mesh,
        out_type=jax.ShapeDtypeStruct((_B, _C // 4), jnp.int32),
        scratch_types=[
            pltpu.VMEM((_EPW,), jnp.int32),
            pltpu.VMEM((_EPW,), jnp.int32),
            pltpu.VMEM((_EPW,), jnp.int32),
            pltpu.VMEM((_EPW, _C // 2), jnp.int32),
            pltpu.VMEM((_EPW, _C // 4), jnp.int32),
            pltpu.SemaphoreType.DMA,
        ],
        compiler_params=pltpu.CompilerParams(needs_layout_passes=False),
    )(_sc_gather_body)
    return run(mpair, idx)


def _mul_body(m_ref, x_ref, o_ref):
    o_ref[...] = x_ref[...] * m_ref[...].astype(jnp.float32)[None]


def kernel(X, idx, masks):
    n = masks.shape[0]
    mpair = lax.bitcast_convert_type(
        masks.view(jnp.uint8).reshape(n // 2, _C // 2, 4), jnp.int32)
    bm_i32 = _sc_gather_masks(mpair, idx.astype(jnp.int32))
    bm = lax.bitcast_convert_type(bm_i32, jnp.uint8).reshape(_B, _C)
    xp = jnp.transpose(X, (2, 3, 0, 1)).reshape(_HW, _B, _C)
    outp = pl.pallas_call(
        _mul_body,
        grid=(_HW // _SLAB,),
        in_specs=[
            pl.BlockSpec((_B, _C), lambda i: (0, 0)),
            pl.BlockSpec((_SLAB, _B, _C), lambda i: (i, 0, 0)),
        ],
        out_specs=pl.BlockSpec((_SLAB, _B, _C), lambda i: (i, 0, 0)),
        out_shape=jax.ShapeDtypeStruct((_HW, _B, _C), jnp.float32),
        compiler_params=pltpu.CompilerParams(
            dimension_semantics=("arbitrary",),
        ),
    )(bm, xp)
    return jnp.transpose(outp.reshape(_H, _W, _B, _C), (2, 3, 0, 1))
